# Initial kernel scaffold; baseline (speedup 1.0000x reference)
import jax
import jax.numpy as jnp
from jax.experimental import pallas as pl


def kernel(x, edge_index, batch, W1, b1, W2, b2, F1w, F1b, F2w, F2b):
    raise NotImplementedError("write your pallas kernel here")


# speedup vs baseline: 10.6126x; 10.6126x over previous
import functools

import jax
import jax.numpy as jnp
from jax import lax
from jax.experimental import pallas as pl
from jax.experimental.pallas import tpu as pltpu
from jax.experimental.pallas import tpu_sc as plsc

N = 50000
E = 800000
F = 64
FQ = 16
FH = 32
G = 512
NC = 2
NS = 16
CH = 128
IB = 200

EPT = E // NS
CPT = 400
PADT = CPT * CH - EPT
ACC_ROWS = NS * 3200

EPT_D = E // (NC * NS)
CPT_D = 196
PADT_D = CPT_D * CH - EPT_D

NPT = N // NS
PCH = NPT // CH + 1
PTAIL = NPT - (PCH - 1) * CH
POOL_ROWS = 640

_mesh = plsc.VectorSubcoreMesh(core_axis_name="c", subcore_axis_name="s")
_sc_params = pltpu.CompilerParams(use_tc_tiling_on_sc=False)


def _fill_rows(rows, n, value):
    v = jnp.full((16,), value, jnp.float32)

    def body(i, _):
        rows[i, pl.ds(0, 16)] = v
        return 0

    lax.fori_loop(0, n, body, 0)


@functools.partial(
    pl.kernel,
    out_type=jax.ShapeDtypeStruct((NC * ACC_ROWS, FQ), jnp.float32),
    mesh=_mesh,
    compiler_params=_sc_params,
    scratch_types=[
        pltpu.VMEM((CPT_D, CH), jnp.int32),
        pltpu.VMEM((CH, FQ), jnp.float32),
        pltpu.VMEM((IB, FQ), jnp.float32),
        pltpu.VMEM_SHARED((ACC_ROWS, FQ), jnp.float32),
    ],
)
def _deg_kernel(dstd, out, didx, vones, zbuf, acc):
    c = lax.axis_index("c")
    s = lax.axis_index("s")
    w = c * NS + s

    _fill_rows(vones, CH, 1.0)
    _fill_rows(zbuf, IB, 0.0)
    pltpu.sync_copy(dstd.at[pl.ds(w * CPT_D, CPT_D)], didx)

    def zacc(i, _):
        pltpu.sync_copy(zbuf, acc.at[pl.ds(s * 3200 + i * IB, IB)])
        return 0

    lax.fori_loop(0, 3200 // IB, zacc, 0)
    plsc.subcore_barrier()

    def body(j, _):
        pltpu.sync_copy(vones, acc.at[didx.at[j]], add=True)
        return 0

    lax.fori_loop(0, CPT_D, body, 0)
    plsc.subcore_barrier()

    def ocp(i, _):
        pltpu.sync_copy(acc.at[pl.ds(s * 3200 + i * IB, IB)], zbuf)
        pltpu.sync_copy(zbuf, out.at[pl.ds(c * ACC_ROWS + s * 3200 + i * IB,
                                           IB)])
        return 0

    lax.fori_loop(0, 3200 // IB, ocp, 0)


@functools.partial(
    pl.kernel,
    out_type=jax.ShapeDtypeStruct((NC * ACC_ROWS, FQ), jnp.float32),
    mesh=_mesh,
    compiler_params=_sc_params,
    scratch_types=[
        pltpu.VMEM((IB, CH), jnp.int32),
        pltpu.VMEM((IB, CH), jnp.int32),
        pltpu.VMEM((CH, FQ), jnp.float32),
        pltpu.VMEM((CH, FQ), jnp.float32),
        pltpu.VMEM_SHARED((ACC_ROWS, FQ), jnp.float32),
        pltpu.SemaphoreType.DMA,
        pltpu.SemaphoreType.DMA,
    ],
)
def _seg_kernel(srcs, dsts, ytab, out, sidx, didx, rows0, rows1, acc,
                gsem0, gsem1):
    c = lax.axis_index("c")
    s = lax.axis_index("s")
    w = c * NS + s

    _fill_rows(rows0, CH, 0.0)

    def zacc(i, _):
        pltpu.sync_copy(rows0, acc.at[pl.ds(s * 3200 + i * CH, CH)])
        return 0

    lax.fori_loop(0, 3200 // CH, zacc, 0)
    plsc.subcore_barrier()

    for b in range(CPT // IB):
        pltpu.sync_copy(srcs.at[pl.ds(w * CPT + b * IB, IB)], sidx)
        pltpu.sync_copy(dsts.at[pl.ds(w * CPT + b * IB, IB)], didx)
        pltpu.async_copy(ytab.at[sidx.at[0]], rows0, gsem0)

        def body(k, _):
            j = 2 * k
            pltpu.async_copy(ytab.at[sidx.at[j + 1]], rows1, gsem1)
            pltpu.make_async_copy(ytab.at[sidx.at[j]], rows0, gsem0).wait()
            pltpu.sync_copy(rows0, acc.at[didx.at[j]], add=True)

            @pl.when(k < IB // 2 - 1)
            def _():
                pltpu.async_copy(ytab.at[sidx.at[j + 2]], rows0, gsem0)

            pltpu.make_async_copy(ytab.at[sidx.at[j + 1]], rows1, gsem1).wait()
            pltpu.sync_copy(rows1, acc.at[didx.at[j + 1]], add=True)
            return 0

        lax.fori_loop(0, IB // 2, body, 0)
    plsc.subcore_barrier()

    def obase(i):
        return c * ACC_ROWS + s * 3200 + i * CH

    def ocp(i, buf, sem):
        pltpu.sync_copy(acc.at[pl.ds(s * 3200 + i * CH, CH)], buf)
        pltpu.async_copy(buf, out.at[pl.ds(obase(i), CH)], sem)

    ocp(0, rows0, gsem0)

    def obody(k, _):
        i = 2 * k
        ocp(i + 1, rows1, gsem1)
        pltpu.make_async_copy(rows0, out.at[pl.ds(obase(i), CH)],
                              gsem0).wait()

        @pl.when(k < 3200 // CH // 2 - 1)
        def _():
            ocp(i + 2, rows0, gsem0)

        pltpu.make_async_copy(rows1, out.at[pl.ds(obase(i + 1), CH)],
                              gsem1).wait()
        return 0

    lax.fori_loop(0, 3200 // CH // 2, obody, 0)
    last = 3200 // CH - 1
    ocp(last, rows0, gsem0)
    pltpu.make_async_copy(rows0, out.at[pl.ds(obase(last), CH)], gsem0).wait()


@functools.partial(
    pl.kernel,
    out_type=(jax.ShapeDtypeStruct((NC * G, FH), jnp.float32),
              jax.ShapeDtypeStruct((G, FQ), jnp.float32)),
    mesh=_mesh,
    compiler_params=_sc_params,
    scratch_types=[
        pltpu.VMEM((PCH, CH), jnp.int32),
        pltpu.VMEM((CH, FH), jnp.float32),
        pltpu.VMEM((CH, FQ), jnp.float32),
        pltpu.VMEM((POOL_ROWS // NS, FH), jnp.float32),
        pltpu.VMEM((POOL_ROWS // NS, FQ), jnp.float32),
        pltpu.VMEM((G // NS, FH), jnp.float32),
        pltpu.VMEM((G // NS, FQ), jnp.float32),
        pltpu.VMEM_SHARED((POOL_ROWS, FH), jnp.float32),
        pltpu.VMEM_SHARED((POOL_ROWS, FQ), jnp.float32),
    ],
)
def _pool_kernel(t2, batchp, sums, cnts, bidx, rows, vones, zp, zq,
                 obuf, obufc, accp, accc):
    c = lax.axis_index("c")
    s = lax.axis_index("s")
    w = c * NS + s
    base = c * N + s * NPT
    stripe = POOL_ROWS // NS
    z16 = jnp.zeros((16,), jnp.float32)

    _fill_rows(vones, CH, 1.0)
    _fill_rows(zp, stripe, 0.0)
    for i in range(stripe):
        zp[i, pl.ds(FQ, FQ)] = z16
    _fill_rows(zq, stripe, 0.0)

    pltpu.sync_copy(batchp.at[pl.ds(w * PCH, PCH)], bidx)
    pltpu.sync_copy(zp, accp.at[pl.ds(s * stripe, stripe)])
    pltpu.sync_copy(zq, accc.at[pl.ds(s * stripe, stripe)])
    plsc.subcore_barrier()

    def body(j, _):
        pltpu.sync_copy(t2.at[pl.ds(base + j * CH, CH)], rows)
        pltpu.sync_copy(rows, accp.at[bidx.at[j]], add=True)

        @pl.when(c == 0)
        def _():
            pltpu.sync_copy(vones, accc.at[bidx.at[j]], add=True)

        return 0

    lax.fori_loop(0, PCH - 1, body, 0)
    pltpu.sync_copy(t2.at[pl.ds(base + NPT - CH, CH)], rows)
    pltpu.sync_copy(rows, accp.at[bidx.at[PCH - 1]], add=True)

    @pl.when(c == 0)
    def _():
        pltpu.sync_copy(vones, accc.at[bidx.at[PCH - 1]], add=True)

    plsc.subcore_barrier()
    gs = G // NS
    pltpu.sync_copy(accp.at[pl.ds(s * gs, gs)], obuf)
    pltpu.sync_copy(obuf, sums.at[pl.ds(c * G + s * gs, gs)])

    @pl.when(c == 0)
    def _():
        pltpu.sync_copy(accc.at[pl.ds(s * gs, gs)], obufc)
        pltpu.sync_copy(obufc, cnts.at[pl.ds(s * gs, gs)])


RB = 400
NBLK = N // RB
SHBLK = ACC_ROWS // RB


def _qsel(y, q):
    return jnp.where(
        q == 0, y[:, 0 * FQ:1 * FQ],
        jnp.where(q == 1, y[:, 1 * FQ:2 * FQ],
                  jnp.where(q == 2, y[:, 2 * FQ:3 * FQ], y[:, 3 * FQ:4 * FQ])))


def _mm1_body(h_ref, w_ref, p0_ref, p1_ref, y_ref, dinv_ref):
    q = pl.program_id(1)
    deg = p0_ref[...] + p1_ref[...] + 1.0
    dinv = 1.0 / jnp.sqrt(deg)
    xw = lax.dot_general(h_ref[...], w_ref[...], (((1,), (1,)), ((), ())),
                         preferred_element_type=jnp.float32)
    y_ref[...] = _qsel(xw * dinv, q)
    dinv_ref[...] = dinv


def _mm2_body(s0_ref, s1_ref, s2_ref, s3_ref, y0_ref, y1_ref, y2_ref, y3_ref,
              dinv_ref, b_ref, w_ref, o_ref):
    q = pl.program_id(1)
    S = jnp.concatenate(
        [s0_ref[...], s1_ref[...], s2_ref[...], s3_ref[...]], axis=1)
    y1 = jnp.concatenate(
        [y0_ref[...], y1_ref[...], y2_ref[...], y3_ref[...]], axis=1)
    dinv = dinv_ref[...]
    h1 = jnp.maximum(dinv * (S + y1) + b_ref[...], 0.0)
    y2 = lax.dot_general(h1, w_ref[...], (((1,), (1,)), ((), ())),
                         preferred_element_type=jnp.float32) * dinv
    o_ref[...] = _qsel(y2, q)


def _post_body(s0_ref, s1_ref, s2_ref, s3_ref, y0_ref, y1_ref, y2_ref, y3_ref,
               dinv_ref, b_ref, t_ref):
    hsel = pl.program_id(1)
    S = jnp.concatenate(
        [s0_ref[...], s1_ref[...], s2_ref[...], s3_ref[...]], axis=1)
    y = jnp.concatenate(
        [y0_ref[...], y1_ref[...], y2_ref[...], y3_ref[...]], axis=1)
    t = jnp.maximum(dinv_ref[...] * (S + y) + b_ref[...], 0.0)
    t_ref[...] = jnp.where(hsel == 0, t[:, :FH], t[:, FH:])


def _head_body(sums_ref, cnts_ref, f1w_ref, f1b_ref, f2w_ref, f2b_ref, o_ref):
    pooled = sums_ref[...] / jnp.maximum(cnts_ref[...], 1.0)
    e = jnp.maximum(
        lax.dot_general(pooled, f1w_ref[...], (((1,), (1,)), ((), ())),
                        preferred_element_type=jnp.float32) + f1b_ref[...],
        0.0)
    e = lax.dot_general(e, f2w_ref[...], (((1,), (1,)), ((), ())),
                        preferred_element_type=jnp.float32) + f2b_ref[...]
    nrm = jnp.sqrt(jnp.sum(e * e, axis=1, keepdims=True))
    o_ref[...] = e / jnp.maximum(nrm, 1e-12)


def _row_spec(width):
    return pl.BlockSpec((RB, width), lambda i, q: (i, 0))


def _seg_q_specs():
    return [pl.BlockSpec((RB, FQ), lambda i, q: (i, 0)),
            pl.BlockSpec((RB, FQ), lambda i, q: (SHBLK + i, 0))]


def _ytab_q_specs():
    return [pl.BlockSpec((RB, FQ), lambda i, q, b=k * NBLK: (b + i, 0))
            for k in range(4)]


_q_out_spec = pl.BlockSpec((RB, FQ), lambda i, q: (q * NBLK + i, 0))
_w_spec = pl.BlockSpec((F, F), lambda i, q: (0, 0))
_b_spec = pl.BlockSpec((1, F), lambda i, q: (0, 0))

_mm1 = pl.pallas_call(
    _mm1_body,
    grid=(NBLK, 4),
    in_specs=[_row_spec(F), _w_spec, _row_spec(1), _row_spec(1)],
    out_specs=[_q_out_spec, pl.BlockSpec((RB, 1), lambda i, q: (i, 0))],
    out_shape=[jax.ShapeDtypeStruct((4 * N, FQ), jnp.float32),
               jax.ShapeDtypeStruct((N, 1), jnp.float32)],
)

_mm2 = pl.pallas_call(
    _mm2_body,
    grid=(NBLK, 4),
    in_specs=_seg_q_specs() + _seg_q_specs() + _ytab_q_specs()
    + [_row_spec(1), _b_spec, _w_spec],
    out_specs=_q_out_spec,
    out_shape=jax.ShapeDtypeStruct((4 * N, FQ), jnp.float32),
)

_post = pl.pallas_call(
    _post_body,
    grid=(NBLK, 2),
    in_specs=_seg_q_specs() + _seg_q_specs() + _ytab_q_specs()
    + [_row_spec(1), _b_spec],
    out_specs=pl.BlockSpec((RB, FH), lambda i, h: (h * NBLK + i, 0)),
    out_shape=jax.ShapeDtypeStruct((NC * N, FH), jnp.float32),
)

_head = pl.pallas_call(
    _head_body,
    out_shape=jax.ShapeDtypeStruct((G, F), jnp.float32),
)


def kernel(x, edge_index, batch, W1, b1, W2, b2, F1w, F1b, F2w, F2b):
    src = edge_index[0].astype(jnp.int32)
    dst = edge_index[1].astype(jnp.int32)
    bat = batch.astype(jnp.int32)

    pad_s = (jnp.arange(NS * PADT, dtype=jnp.int32) * 4099) % (NC * N)
    pad_s = pad_s.reshape(NS, PADT)
    pad_d = N + (jnp.arange(NS * PADT, dtype=jnp.int32) % (ACC_ROWS - N))
    pad_d = pad_d.reshape(NS, PADT)
    s_t = src.reshape(NS, EPT)
    d_t = dst.reshape(NS, EPT)
    s0 = jnp.concatenate([s_t, pad_s], axis=1)
    s1 = jnp.concatenate([s_t + N, pad_s], axis=1)
    srcs_a = jnp.concatenate([s0, s1], axis=0).reshape(NC * NS * CPT, CH)
    srcs_b = srcs_a + 2 * N
    d0 = jnp.concatenate([d_t, pad_d], axis=1)
    dsts = jnp.concatenate([d0, d0], axis=0).reshape(NC * NS * CPT, CH)

    pad_dd = N + (jnp.arange(NC * NS * PADT_D, dtype=jnp.int32)
                  % (ACC_ROWS - N)).reshape(NC * NS, PADT_D)
    dstd = jnp.concatenate([dst.reshape(NC * NS, EPT_D), pad_dd],
                           axis=1).reshape(NC * NS * CPT_D, CH)

    pad_b = G + (jnp.arange(NS * (CH - PTAIL), dtype=jnp.int32)
                 % (POOL_ROWS - G)).reshape(NS, CH - PTAIL)
    bt = bat.reshape(NS, NPT)
    b_full = bt[:, :(PCH - 1) * CH].reshape(NS, PCH - 1, CH)
    b_tail = jnp.concatenate([pad_b, bt[:, (PCH - 1) * CH:]], axis=1)
    b_t = jnp.concatenate([b_full, b_tail[:, None, :]], axis=1)
    batchp = jnp.concatenate([b_t, b_t], axis=0).reshape(NC * NS * PCH, CH)

    degp = _deg_kernel(dstd)
    p0 = degp[:N, :1]
    p1 = degp[ACC_ROWS:ACC_ROWS + N, :1]

    h = x[:, 1:]
    ytab1, dinv = _mm1(h, W1, p0, p1)
    S1a = _seg_kernel(srcs_a, dsts, ytab1)
    S1b = _seg_kernel(srcs_b, dsts, ytab1)
    ytab2 = _mm2(S1a, S1a, S1b, S1b, ytab1, ytab1, ytab1, ytab1,
                 dinv, b1.reshape(1, F), W2)
    S2a = _seg_kernel(srcs_a, dsts, ytab2)
    S2b = _seg_kernel(srcs_b, dsts, ytab2)
    t2 = _post(S2a, S2a, S2b, S2b, ytab2, ytab2, ytab2, ytab2,
               dinv, b2.reshape(1, F))
    sums, cnts = _pool_kernel(t2, batchp)
    psum = jnp.concatenate([sums[:G], sums[G:]], axis=1)
    return _head(psum, cnts[:, :1], F1w, F1b.reshape(1, F),
                 F2w, F2b.reshape(1, F))


# speedup vs baseline: 12.1952x; 1.1491x over previous
import functools

import jax
import jax.numpy as jnp
from jax import lax
from jax.experimental import pallas as pl
from jax.experimental.pallas import tpu as pltpu
from jax.experimental.pallas import tpu_sc as plsc

N = 50000
E = 800000
F = 64
FQ = 16
FH = 32
G = 512
NC = 2
NS = 16
CH = 128
IB = 200

EPT = E // NS
CPT = 400
PADT = CPT * CH - EPT
ACC_ROWS = NS * 3200

EPT_D = E // (NC * NS)
CPT_D = 196
PADT_D = CPT_D * CH - EPT_D

NPT = N // NS
PCH = NPT // CH + 1
PTAIL = NPT - (PCH - 1) * CH
POOL_ROWS = 640

_mesh = plsc.VectorSubcoreMesh(core_axis_name="c", subcore_axis_name="s")
_sc_params = pltpu.CompilerParams(use_tc_tiling_on_sc=False)


def _fill_rows(rows, n, value):
    v = jnp.full((16,), value, jnp.float32)

    def body(i, _):
        rows[i, pl.ds(0, 16)] = v
        return 0

    lax.fori_loop(0, n, body, 0)


@functools.partial(
    pl.kernel,
    out_type=jax.ShapeDtypeStruct((NC * ACC_ROWS, FQ), jnp.float32),
    mesh=_mesh,
    compiler_params=_sc_params,
    scratch_types=[
        pltpu.VMEM((CPT_D, CH), jnp.int32),
        pltpu.VMEM((CH, FQ), jnp.float32),
        pltpu.VMEM((IB, FQ), jnp.float32),
        pltpu.VMEM_SHARED((ACC_ROWS, FQ), jnp.float32),
    ],
)
def _deg_kernel(dstd, out, didx, vones, zbuf, acc):
    c = lax.axis_index("c")
    s = lax.axis_index("s")
    w = c * NS + s

    _fill_rows(vones, CH, 1.0)
    _fill_rows(zbuf, IB, 0.0)
    pltpu.sync_copy(dstd.at[pl.ds(w * CPT_D, CPT_D)], didx)

    def zacc(i, _):
        pltpu.sync_copy(zbuf, acc.at[pl.ds(s * 3200 + i * IB, IB)])
        return 0

    lax.fori_loop(0, 3200 // IB, zacc, 0)
    plsc.subcore_barrier()

    def body(j, _):
        pltpu.sync_copy(vones, acc.at[didx.at[j]], add=True)
        return 0

    lax.fori_loop(0, CPT_D, body, 0)
    plsc.subcore_barrier()

    def ocp(i, _):
        pltpu.sync_copy(acc.at[pl.ds(s * 3200 + i * IB, IB)], zbuf)
        pltpu.sync_copy(zbuf, out.at[pl.ds(c * ACC_ROWS + s * 3200 + i * IB,
                                           IB)])
        return 0

    lax.fori_loop(0, 3200 // IB, ocp, 0)


@functools.partial(
    pl.kernel,
    out_type=jax.ShapeDtypeStruct((NC * ACC_ROWS, FQ), jnp.float32),
    mesh=_mesh,
    compiler_params=_sc_params,
    scratch_types=[
        pltpu.VMEM((IB, CH), jnp.int32),
        pltpu.VMEM((IB, CH), jnp.int32),
        pltpu.VMEM((CH, FQ), jnp.float32),
        pltpu.VMEM((CH, FQ), jnp.float32),
        pltpu.VMEM_SHARED((ACC_ROWS, FQ), jnp.float32),
        pltpu.SemaphoreType.DMA,
        pltpu.SemaphoreType.DMA,
    ],
)
def _seg_kernel(srcs, dsts, ytab, out, sidx, didx, rows0, rows1, acc,
                gsem0, gsem1):
    c = lax.axis_index("c")
    s = lax.axis_index("s")
    w = c * NS + s

    _fill_rows(rows0, CH, 0.0)

    def zacc(i, _):
        pltpu.sync_copy(rows0, acc.at[pl.ds(s * 3200 + i * CH, CH)])
        return 0

    lax.fori_loop(0, 3200 // CH, zacc, 0)
    plsc.subcore_barrier()

    for b in range(CPT // IB):
        pltpu.sync_copy(srcs.at[pl.ds(w * CPT + b * IB, IB)], sidx)
        pltpu.sync_copy(dsts.at[pl.ds(w * CPT + b * IB, IB)], didx)
        pltpu.async_copy(ytab.at[sidx.at[0]], rows0, gsem0)

        def body(k, _):
            j = 2 * k
            pltpu.async_copy(ytab.at[sidx.at[j + 1]], rows1, gsem1)
            pltpu.make_async_copy(ytab.at[sidx.at[j]], rows0, gsem0).wait()
            pltpu.sync_copy(rows0, acc.at[didx.at[j]], add=True)

            @pl.when(k < IB // 2 - 1)
            def _():
                pltpu.async_copy(ytab.at[sidx.at[j + 2]], rows0, gsem0)

            pltpu.make_async_copy(ytab.at[sidx.at[j + 1]], rows1, gsem1).wait()
            pltpu.sync_copy(rows1, acc.at[didx.at[j + 1]], add=True)
            return 0

        lax.fori_loop(0, IB // 2, body, 0)
    plsc.subcore_barrier()

    def obase(i):
        return c * ACC_ROWS + s * 3200 + i * CH

    def ocp(i, buf, sem):
        pltpu.sync_copy(acc.at[pl.ds(s * 3200 + i * CH, CH)], buf)
        pltpu.async_copy(buf, out.at[pl.ds(obase(i), CH)], sem)

    ocp(0, rows0, gsem0)

    def obody(k, _):
        i = 2 * k
        ocp(i + 1, rows1, gsem1)
        pltpu.make_async_copy(rows0, out.at[pl.ds(obase(i), CH)],
                              gsem0).wait()

        @pl.when(k < 3200 // CH // 2 - 1)
        def _():
            ocp(i + 2, rows0, gsem0)

        pltpu.make_async_copy(rows1, out.at[pl.ds(obase(i + 1), CH)],
                              gsem1).wait()
        return 0

    lax.fori_loop(0, 3200 // CH // 2, obody, 0)
    last = 3200 // CH - 1
    ocp(last, rows0, gsem0)
    pltpu.make_async_copy(rows0, out.at[pl.ds(obase(last), CH)], gsem0).wait()


@functools.partial(
    pl.kernel,
    out_type=(jax.ShapeDtypeStruct((NC * G, FH), jnp.float32),
              jax.ShapeDtypeStruct((G, FQ), jnp.float32)),
    mesh=_mesh,
    compiler_params=_sc_params,
    scratch_types=[
        pltpu.VMEM((PCH, CH), jnp.int32),
        pltpu.VMEM((CH, FH), jnp.float32),
        pltpu.VMEM((CH, FQ), jnp.float32),
        pltpu.VMEM((POOL_ROWS // NS, FH), jnp.float32),
        pltpu.VMEM((POOL_ROWS // NS, FQ), jnp.float32),
        pltpu.VMEM((G // NS, FH), jnp.float32),
        pltpu.VMEM((G // NS, FQ), jnp.float32),
        pltpu.VMEM_SHARED((POOL_ROWS, FH), jnp.float32),
        pltpu.VMEM_SHARED((POOL_ROWS, FQ), jnp.float32),
    ],
)
def _pool_kernel(t2, batchp, sums, cnts, bidx, rows, vones, zp, zq,
                 obuf, obufc, accp, accc):
    c = lax.axis_index("c")
    s = lax.axis_index("s")
    w = c * NS + s
    base = c * N + s * NPT
    stripe = POOL_ROWS // NS
    z16 = jnp.zeros((16,), jnp.float32)

    _fill_rows(vones, CH, 1.0)
    _fill_rows(zp, stripe, 0.0)
    for i in range(stripe):
        zp[i, pl.ds(FQ, FQ)] = z16
    _fill_rows(zq, stripe, 0.0)

    pltpu.sync_copy(batchp.at[pl.ds(w * PCH, PCH)], bidx)
    pltpu.sync_copy(zp, accp.at[pl.ds(s * stripe, stripe)])
    pltpu.sync_copy(zq, accc.at[pl.ds(s * stripe, stripe)])
    plsc.subcore_barrier()

    def body(j, _):
        pltpu.sync_copy(t2.at[pl.ds(base + j * CH, CH)], rows)
        pltpu.sync_copy(rows, accp.at[bidx.at[j]], add=True)

        @pl.when(c == 0)
        def _():
            pltpu.sync_copy(vones, accc.at[bidx.at[j]], add=True)

        return 0

    lax.fori_loop(0, PCH - 1, body, 0)
    pltpu.sync_copy(t2.at[pl.ds(base + NPT - CH, CH)], rows)
    pltpu.sync_copy(rows, accp.at[bidx.at[PCH - 1]], add=True)

    @pl.when(c == 0)
    def _():
        pltpu.sync_copy(vones, accc.at[bidx.at[PCH - 1]], add=True)

    plsc.subcore_barrier()
    gs = G // NS
    pltpu.sync_copy(accp.at[pl.ds(s * gs, gs)], obuf)
    pltpu.sync_copy(obuf, sums.at[pl.ds(c * G + s * gs, gs)])

    @pl.when(c == 0)
    def _():
        pltpu.sync_copy(accc.at[pl.ds(s * gs, gs)], obufc)
        pltpu.sync_copy(obufc, cnts.at[pl.ds(s * gs, gs)])


RB = 400
NBLK = N // RB


def _mm1_body(h_ref, w_ref, p0_ref, p1_ref, y_ref, dinv_ref):
    deg = p0_ref[...] + p1_ref[...] + 1.0
    dinv = 1.0 / jnp.sqrt(deg)
    xw = lax.dot_general(h_ref[...], w_ref[...], (((1,), (1,)), ((), ())),
                         preferred_element_type=jnp.float32)
    y_ref[...] = xw * dinv
    dinv_ref[...] = dinv


def _mm2_body(s_ref, y_ref, dinv_ref, b_ref, w_ref, o_ref):
    dinv = dinv_ref[...]
    h1 = jnp.maximum(dinv * (s_ref[...] + y_ref[...]) + b_ref[...], 0.0)
    o_ref[...] = lax.dot_general(h1, w_ref[...], (((1,), (1,)), ((), ())),
                                 preferred_element_type=jnp.float32) * dinv


def _post_body(s_ref, y_ref, dinv_ref, b_ref, t_ref):
    t_ref[...] = jnp.maximum(
        dinv_ref[...] * (s_ref[...] + y_ref[...]) + b_ref[...], 0.0)


def _head_body(sums_ref, cnts_ref, f1w_ref, f1b_ref, f2w_ref, f2b_ref, o_ref):
    pooled = sums_ref[...] / jnp.maximum(cnts_ref[...], 1.0)
    e = jnp.maximum(
        lax.dot_general(pooled, f1w_ref[...], (((1,), (1,)), ((), ())),
                        preferred_element_type=jnp.float32) + f1b_ref[...],
        0.0)
    e = lax.dot_general(e, f2w_ref[...], (((1,), (1,)), ((), ())),
                        preferred_element_type=jnp.float32) + f2b_ref[...]
    nrm = jnp.sqrt(jnp.sum(e * e, axis=1, keepdims=True))
    o_ref[...] = e / jnp.maximum(nrm, 1e-12)


def _row_spec(width):
    return pl.BlockSpec((RB, width), lambda i: (i, 0))


_w_spec = pl.BlockSpec((F, F), lambda i: (0, 0))
_b_spec = pl.BlockSpec((1, F), lambda i: (0, 0))

_mm1 = pl.pallas_call(
    _mm1_body,
    grid=(NBLK,),
    in_specs=[_row_spec(F), _w_spec, _row_spec(1), _row_spec(1)],
    out_specs=[_row_spec(F), _row_spec(1)],
    out_shape=[jax.ShapeDtypeStruct((N, F), jnp.float32),
               jax.ShapeDtypeStruct((N, 1), jnp.float32)],
)

_mm2 = pl.pallas_call(
    _mm2_body,
    grid=(NBLK,),
    in_specs=[_row_spec(F), _row_spec(F), _row_spec(1), _b_spec, _w_spec],
    out_specs=_row_spec(F),
    out_shape=jax.ShapeDtypeStruct((N, F), jnp.float32),
)

_post = pl.pallas_call(
    _post_body,
    grid=(NBLK,),
    in_specs=[_row_spec(F), _row_spec(F), _row_spec(1), _b_spec],
    out_specs=_row_spec(F),
    out_shape=jax.ShapeDtypeStruct((N, F), jnp.float32),
)

_head = pl.pallas_call(
    _head_body,
    out_shape=jax.ShapeDtypeStruct((G, F), jnp.float32),
)


def _to_quarters(y):
    return jnp.concatenate([y[:, k * FQ:(k + 1) * FQ] for k in range(4)],
                           axis=0)


def _from_seg(Sa, Sb):
    return jnp.concatenate(
        [Sa[:N], Sa[ACC_ROWS:ACC_ROWS + N],
         Sb[:N], Sb[ACC_ROWS:ACC_ROWS + N]], axis=1)


def kernel(x, edge_index, batch, W1, b1, W2, b2, F1w, F1b, F2w, F2b):
    src = edge_index[0].astype(jnp.int32)
    dst = edge_index[1].astype(jnp.int32)
    bat = batch.astype(jnp.int32)

    pad_s = (jnp.arange(NS * PADT, dtype=jnp.int32) * 4099) % (NC * N)
    pad_s = pad_s.reshape(NS, PADT)
    pad_d = N + (jnp.arange(NS * PADT, dtype=jnp.int32) % (ACC_ROWS - N))
    pad_d = pad_d.reshape(NS, PADT)
    s_t = src.reshape(NS, EPT)
    d_t = dst.reshape(NS, EPT)
    s0 = jnp.concatenate([s_t, pad_s], axis=1)
    s1 = jnp.concatenate([s_t + N, pad_s], axis=1)
    srcs_a = jnp.concatenate([s0, s1], axis=0).reshape(NC * NS * CPT, CH)
    srcs_b = srcs_a + 2 * N
    d0 = jnp.concatenate([d_t, pad_d], axis=1)
    dsts = jnp.concatenate([d0, d0], axis=0).reshape(NC * NS * CPT, CH)

    pad_dd = N + (jnp.arange(NC * NS * PADT_D, dtype=jnp.int32)
                  % (ACC_ROWS - N)).reshape(NC * NS, PADT_D)
    dstd = jnp.concatenate([dst.reshape(NC * NS, EPT_D), pad_dd],
                           axis=1).reshape(NC * NS * CPT_D, CH)

    pad_b = G + (jnp.arange(NS * (CH - PTAIL), dtype=jnp.int32)
                 % (POOL_ROWS - G)).reshape(NS, CH - PTAIL)
    bt = bat.reshape(NS, NPT)
    b_full = bt[:, :(PCH - 1) * CH].reshape(NS, PCH - 1, CH)
    b_tail = jnp.concatenate([pad_b, bt[:, (PCH - 1) * CH:]], axis=1)
    b_t = jnp.concatenate([b_full, b_tail[:, None, :]], axis=1)
    batchp = jnp.concatenate([b_t, b_t], axis=0).reshape(NC * NS * PCH, CH)

    degp = _deg_kernel(dstd)
    p0 = degp[:N, :1]
    p1 = degp[ACC_ROWS:ACC_ROWS + N, :1]

    h = x[:, 1:]
    y1, dinv = _mm1(h, W1, p0, p1)
    ytab1 = _to_quarters(y1)
    S1a = _seg_kernel(srcs_a, dsts, ytab1)
    S1b = _seg_kernel(srcs_b, dsts, ytab1)
    y2 = _mm2(_from_seg(S1a, S1b), y1, dinv, b1.reshape(1, F), W2)
    ytab2 = _to_quarters(y2)
    S2a = _seg_kernel(srcs_a, dsts, ytab2)
    S2b = _seg_kernel(srcs_b, dsts, ytab2)
    t = _post(_from_seg(S2a, S2b), y2, dinv, b2.reshape(1, F))
    t2 = jnp.concatenate([t[:, :FH], t[:, FH:]], axis=0)
    sums, cnts = _pool_kernel(t2, batchp)
    psum = jnp.concatenate([sums[:G], sums[G:]], axis=1)
    return _head(psum, cnts[:, :1], F1w, F1b.reshape(1, F),
                 F2w, F2b.reshape(1, F))


# speedup vs baseline: 12.2350x; 1.0033x over previous
import functools

import jax
import jax.numpy as jnp
from jax import lax
from jax.experimental import pallas as pl
from jax.experimental.pallas import tpu as pltpu
from jax.experimental.pallas import tpu_sc as plsc

N = 50000
E = 800000
F = 64
FQ = 16
FH = 32
G = 512
NC = 2
NS = 16
CH = 128
IB = 200

EPT = E // NS
CPT = 400
PADT = CPT * CH - EPT
ACC_ROWS = NS * 3200

EPT_D = E // (NC * NS)
CPT_D = 196
PADT_D = CPT_D * CH - EPT_D

NPT = N // NS
PCH = NPT // CH + 1
PTAIL = NPT - (PCH - 1) * CH
POOL_ROWS = 640

_mesh = plsc.VectorSubcoreMesh(core_axis_name="c", subcore_axis_name="s")
_sc_params = pltpu.CompilerParams(use_tc_tiling_on_sc=False)


def _fill_rows(rows, n, value):
    v = jnp.full((16,), value, jnp.float32)

    def body(i, _):
        rows[i, pl.ds(0, 16)] = v
        return 0

    lax.fori_loop(0, n, body, 0)


@functools.partial(
    pl.kernel,
    out_type=jax.ShapeDtypeStruct((NC * ACC_ROWS, FQ), jnp.float32),
    mesh=_mesh,
    compiler_params=_sc_params,
    scratch_types=[
        pltpu.VMEM((CPT_D, CH), jnp.int32),
        pltpu.VMEM((CH, FQ), jnp.float32),
        pltpu.VMEM((IB, FQ), jnp.float32),
        pltpu.VMEM_SHARED((ACC_ROWS, FQ), jnp.float32),
    ],
)
def _deg_kernel(dstd, out, didx, vones, zbuf, acc):
    c = lax.axis_index("c")
    s = lax.axis_index("s")
    w = c * NS + s

    _fill_rows(vones, CH, 1.0)
    _fill_rows(zbuf, IB, 0.0)
    pltpu.sync_copy(dstd.at[pl.ds(w * CPT_D, CPT_D)], didx)

    def zacc(i, _):
        pltpu.sync_copy(zbuf, acc.at[pl.ds(s * 3200 + i * IB, IB)])
        return 0

    lax.fori_loop(0, 3200 // IB, zacc, 0)
    plsc.subcore_barrier()

    def body(j, _):
        pltpu.sync_copy(vones, acc.at[didx.at[j]], add=True)
        return 0

    lax.fori_loop(0, CPT_D, body, 0)
    plsc.subcore_barrier()

    def ocp(i, _):
        pltpu.sync_copy(acc.at[pl.ds(s * 3200 + i * IB, IB)], zbuf)
        pltpu.sync_copy(zbuf, out.at[pl.ds(c * ACC_ROWS + s * 3200 + i * IB,
                                           IB)])
        return 0

    lax.fori_loop(0, 3200 // IB, ocp, 0)


@functools.partial(
    pl.kernel,
    out_type=jax.ShapeDtypeStruct((NC * ACC_ROWS, FQ), jnp.float32),
    mesh=_mesh,
    compiler_params=_sc_params,
    scratch_types=[
        pltpu.VMEM((IB, CH), jnp.int32),
        pltpu.VMEM((IB, CH), jnp.int32),
        pltpu.VMEM((CH, FQ), jnp.float32),
        pltpu.VMEM((CH, FQ), jnp.float32),
        pltpu.VMEM_SHARED((ACC_ROWS, FQ), jnp.float32),
        pltpu.SemaphoreType.DMA,
        pltpu.SemaphoreType.DMA,
    ],
)
def _seg_kernel(srcs, dsts, ytab, out, sidx, didx, rows0, rows1, acc,
                gsem0, gsem1):
    c = lax.axis_index("c")
    s = lax.axis_index("s")
    w = c * NS + s

    _fill_rows(rows0, CH, 0.0)

    def zacc(i, _):
        pltpu.sync_copy(rows0, acc.at[pl.ds(s * 3200 + i * CH, CH)])
        return 0

    lax.fori_loop(0, 3200 // CH, zacc, 0)
    plsc.subcore_barrier()

    for b in range(CPT // IB):
        pltpu.sync_copy(srcs.at[pl.ds(w * CPT + b * IB, IB)], sidx)
        pltpu.sync_copy(dsts.at[pl.ds(w * CPT + b * IB, IB)], didx)
        pltpu.async_copy(ytab.at[sidx.at[0]], rows0, gsem0)

        def body(k, _):
            j = 2 * k
            pltpu.async_copy(ytab.at[sidx.at[j + 1]], rows1, gsem1)
            pltpu.make_async_copy(ytab.at[sidx.at[j]], rows0, gsem0).wait()
            pltpu.sync_copy(rows0, acc.at[didx.at[j]], add=True)

            @pl.when(k < IB // 2 - 1)
            def _():
                pltpu.async_copy(ytab.at[sidx.at[j + 2]], rows0, gsem0)

            pltpu.make_async_copy(ytab.at[sidx.at[j + 1]], rows1, gsem1).wait()
            pltpu.sync_copy(rows1, acc.at[didx.at[j + 1]], add=True)
            return 0

        lax.fori_loop(0, IB // 2, body, 0)
    plsc.subcore_barrier()

    def obase(i):
        return c * ACC_ROWS + s * 3200 + i * CH

    def ocp(i, buf, sem):
        pltpu.sync_copy(acc.at[pl.ds(s * 3200 + i * CH, CH)], buf)
        pltpu.async_copy(buf, out.at[pl.ds(obase(i), CH)], sem)

    ocp(0, rows0, gsem0)

    def obody(k, _):
        i = 2 * k
        ocp(i + 1, rows1, gsem1)
        pltpu.make_async_copy(rows0, out.at[pl.ds(obase(i), CH)],
                              gsem0).wait()

        @pl.when(k < 3200 // CH // 2 - 1)
        def _():
            ocp(i + 2, rows0, gsem0)

        pltpu.make_async_copy(rows1, out.at[pl.ds(obase(i + 1), CH)],
                              gsem1).wait()
        return 0

    lax.fori_loop(0, 3200 // CH // 2, obody, 0)
    last = 3200 // CH - 1
    ocp(last, rows0, gsem0)
    pltpu.make_async_copy(rows0, out.at[pl.ds(obase(last), CH)], gsem0).wait()


@functools.partial(
    pl.kernel,
    out_type=(jax.ShapeDtypeStruct((NC * G, FH), jnp.float32),
              jax.ShapeDtypeStruct((G, FQ), jnp.float32)),
    mesh=_mesh,
    compiler_params=_sc_params,
    scratch_types=[
        pltpu.VMEM((PCH, CH), jnp.int32),
        pltpu.VMEM((CH, FH), jnp.float32),
        pltpu.VMEM((CH, FQ), jnp.float32),
        pltpu.VMEM((POOL_ROWS // NS, FH), jnp.float32),
        pltpu.VMEM((POOL_ROWS // NS, FQ), jnp.float32),
        pltpu.VMEM((G // NS, FH), jnp.float32),
        pltpu.VMEM((G // NS, FQ), jnp.float32),
        pltpu.VMEM_SHARED((POOL_ROWS, FH), jnp.float32),
        pltpu.VMEM_SHARED((POOL_ROWS, FQ), jnp.float32),
    ],
)
def _pool_kernel(t2, batchp, sums, cnts, bidx, rows, vones, zp, zq,
                 obuf, obufc, accp, accc):
    c = lax.axis_index("c")
    s = lax.axis_index("s")
    w = c * NS + s
    base = c * N + s * NPT
    stripe = POOL_ROWS // NS
    z16 = jnp.zeros((16,), jnp.float32)

    _fill_rows(vones, CH, 1.0)
    _fill_rows(zp, stripe, 0.0)
    for i in range(stripe):
        zp[i, pl.ds(FQ, FQ)] = z16
    _fill_rows(zq, stripe, 0.0)

    pltpu.sync_copy(batchp.at[pl.ds(w * PCH, PCH)], bidx)
    pltpu.sync_copy(zp, accp.at[pl.ds(s * stripe, stripe)])
    pltpu.sync_copy(zq, accc.at[pl.ds(s * stripe, stripe)])
    plsc.subcore_barrier()

    def body(j, _):
        pltpu.sync_copy(t2.at[pl.ds(base + j * CH, CH)], rows)
        pltpu.sync_copy(rows, accp.at[bidx.at[j]], add=True)

        @pl.when(c == 0)
        def _():
            pltpu.sync_copy(vones, accc.at[bidx.at[j]], add=True)

        return 0

    lax.fori_loop(0, PCH - 1, body, 0)
    pltpu.sync_copy(t2.at[pl.ds(base + NPT - CH, CH)], rows)
    pltpu.sync_copy(rows, accp.at[bidx.at[PCH - 1]], add=True)

    @pl.when(c == 0)
    def _():
        pltpu.sync_copy(vones, accc.at[bidx.at[PCH - 1]], add=True)

    plsc.subcore_barrier()
    gs = G // NS
    pltpu.sync_copy(accp.at[pl.ds(s * gs, gs)], obuf)
    pltpu.sync_copy(obuf, sums.at[pl.ds(c * G + s * gs, gs)])

    @pl.when(c == 0)
    def _():
        pltpu.sync_copy(accc.at[pl.ds(s * gs, gs)], obufc)
        pltpu.sync_copy(obufc, cnts.at[pl.ds(s * gs, gs)])


RB = 400
NBLK = N // RB


def _mm1_body(h_ref, w_ref, p0_ref, p1_ref, y_ref, dinv_ref):
    deg = p0_ref[...] + p1_ref[...] + 1.0
    dinv = 1.0 / jnp.sqrt(deg)
    xw = lax.dot_general(h_ref[...], w_ref[...], (((1,), (1,)), ((), ())),
                         preferred_element_type=jnp.float32)
    y_ref[...] = xw * dinv
    dinv_ref[...] = dinv


def _mm2_body(s_ref, y_ref, dinv_ref, b_ref, w_ref, o_ref):
    dinv = dinv_ref[...]
    h1 = jnp.maximum(dinv * (s_ref[...] + y_ref[...]) + b_ref[...], 0.0)
    o_ref[...] = lax.dot_general(h1, w_ref[...], (((1,), (1,)), ((), ())),
                                 preferred_element_type=jnp.float32) * dinv


def _post_body(s_ref, y_ref, dinv_ref, b_ref, t_ref):
    t_ref[...] = jnp.maximum(
        dinv_ref[...] * (s_ref[...] + y_ref[...]) + b_ref[...], 0.0)


def _head_body(sums_ref, cnts_ref, f1w_ref, f1b_ref, f2w_ref, f2b_ref, o_ref):
    pooled = sums_ref[...] / jnp.maximum(cnts_ref[...], 1.0)
    e = jnp.maximum(
        lax.dot_general(pooled, f1w_ref[...], (((1,), (1,)), ((), ())),
                        preferred_element_type=jnp.float32) + f1b_ref[...],
        0.0)
    e = lax.dot_general(e, f2w_ref[...], (((1,), (1,)), ((), ())),
                        preferred_element_type=jnp.float32) + f2b_ref[...]
    nrm = jnp.sqrt(jnp.sum(e * e, axis=1, keepdims=True))
    o_ref[...] = e / jnp.maximum(nrm, 1e-12)


def _row_spec(width):
    return pl.BlockSpec((RB, width), lambda i: (i, 0))


_w_spec = pl.BlockSpec((F, F), lambda i: (0, 0))
_b_spec = pl.BlockSpec((1, F), lambda i: (0, 0))

_mm1 = pl.pallas_call(
    _mm1_body,
    grid=(NBLK,),
    in_specs=[_row_spec(F), _w_spec, _row_spec(1), _row_spec(1)],
    out_specs=[_row_spec(F), _row_spec(1)],
    out_shape=[jax.ShapeDtypeStruct((N, F), jnp.float32),
               jax.ShapeDtypeStruct((N, 1), jnp.float32)],
)

_mm2 = pl.pallas_call(
    _mm2_body,
    grid=(NBLK,),
    in_specs=[_row_spec(F), _row_spec(F), _row_spec(1), _b_spec, _w_spec],
    out_specs=_row_spec(F),
    out_shape=jax.ShapeDtypeStruct((N, F), jnp.float32),
)

_post = pl.pallas_call(
    _post_body,
    grid=(NBLK,),
    in_specs=[_row_spec(F), _row_spec(F), _row_spec(1), _b_spec],
    out_specs=_row_spec(F),
    out_shape=jax.ShapeDtypeStruct((N, F), jnp.float32),
)

_head = pl.pallas_call(
    _head_body,
    out_shape=jax.ShapeDtypeStruct((G, F), jnp.float32),
)


def _to_quarters(y):
    return jnp.concatenate([y[:, k * FQ:(k + 1) * FQ] for k in range(4)],
                           axis=0)


def _from_seg(Sa, Sb):
    return jnp.concatenate(
        [Sa[:N], Sa[ACC_ROWS:ACC_ROWS + N],
         Sb[:N], Sb[ACC_ROWS:ACC_ROWS + N]], axis=1)


def kernel(x, edge_index, batch, W1, b1, W2, b2, F1w, F1b, F2w, F2b):
    src = edge_index[0].astype(jnp.int32)
    dst = edge_index[1].astype(jnp.int32)
    bat = batch.astype(jnp.int32)

    pad_s = (jnp.arange(NS * PADT, dtype=jnp.int32) * 4099) % (NC * N)
    pad_s = pad_s.reshape(NS, PADT)
    pad_d = N + (jnp.arange(NS * PADT, dtype=jnp.int32) % (ACC_ROWS - N))
    pad_d = pad_d.reshape(NS, PADT)
    s_t = src.reshape(NS, EPT)
    d_t = dst.reshape(NS, EPT)
    s0 = jnp.concatenate([s_t, pad_s], axis=1)
    s1 = jnp.concatenate([s_t + N, pad_s], axis=1)
    srcs_a = jnp.concatenate([s0, s1], axis=0).reshape(NC * NS * CPT, CH)
    s2 = jnp.concatenate([s_t + 2 * N, pad_s], axis=1)
    s3 = jnp.concatenate([s_t + 3 * N, pad_s], axis=1)
    srcs_b = jnp.concatenate([s2, s3], axis=0).reshape(NC * NS * CPT, CH)
    d0 = jnp.concatenate([d_t, pad_d], axis=1)
    dsts = jnp.concatenate([d0, d0], axis=0).reshape(NC * NS * CPT, CH)

    pad_dd = N + (jnp.arange(NC * NS * PADT_D, dtype=jnp.int32)
                  % (ACC_ROWS - N)).reshape(NC * NS, PADT_D)
    dstd = jnp.concatenate([dst.reshape(NC * NS, EPT_D), pad_dd],
                           axis=1).reshape(NC * NS * CPT_D, CH)

    pad_b = G + (jnp.arange(NS * (CH - PTAIL), dtype=jnp.int32)
                 % (POOL_ROWS - G)).reshape(NS, CH - PTAIL)
    bt = bat.reshape(NS, NPT)
    b_full = bt[:, :(PCH - 1) * CH].reshape(NS, PCH - 1, CH)
    b_tail = jnp.concatenate([pad_b, bt[:, (PCH - 1) * CH:]], axis=1)
    b_t = jnp.concatenate([b_full, b_tail[:, None, :]], axis=1)
    batchp = jnp.concatenate([b_t, b_t], axis=0).reshape(NC * NS * PCH, CH)

    degp = _deg_kernel(dstd)
    p0 = degp[:N, :1]
    p1 = degp[ACC_ROWS:ACC_ROWS + N, :1]

    h = x[:, 1:]
    y1, dinv = _mm1(h, W1, p0, p1)
    ytab1 = _to_quarters(y1)
    S1a = _seg_kernel(srcs_a, dsts, ytab1)
    S1b = _seg_kernel(srcs_b, dsts, ytab1)
    y2 = _mm2(_from_seg(S1a, S1b), y1, dinv, b1.reshape(1, F), W2)
    ytab2 = _to_quarters(y2)
    S2a = _seg_kernel(srcs_a, dsts, ytab2)
    S2b = _seg_kernel(srcs_b, dsts, ytab2)
    t = _post(_from_seg(S2a, S2b), y2, dinv, b2.reshape(1, F))
    t2 = jnp.concatenate([t[:, :FH], t[:, FH:]], axis=0)
    sums, cnts = _pool_kernel(t2, batchp)
    psum = jnp.concatenate([sums[:G], sums[G:]], axis=1)
    return _head(psum, cnts[:, :1], F1w, F1b.reshape(1, F),
                 F2w, F2b.reshape(1, F))
